# Initial kernel scaffold; baseline (speedup 1.0000x reference)
import jax
import jax.numpy as jnp
from jax.experimental import pallas as pl


def kernel(vectors, x):
    raise NotImplementedError("write your pallas kernel here")


# speedup vs baseline: 6.5988x; 6.5988x over previous
import functools

import jax
import jax.numpy as jnp
from jax import lax
from jax.experimental import pallas as pl
from jax.experimental.pallas import tpu as pltpu
from jax.experimental.pallas import tpu_sc as plsc

B = 4096
L = 50
D = 64
NW = 32
BPW = B // NW
RPC = 2
NCH = BPW // RPC
IPC = RPC * L
NJ = D // 16

_mesh = plsc.VectorSubcoreMesh(core_axis_name="c", subcore_axis_name="s")


@functools.partial(
    pl.kernel,
    mesh=_mesh,
    compiler_params=pltpu.CompilerParams(use_tc_tiling_on_sc=False),
    out_type=jax.ShapeDtypeStruct((B, D), jnp.float32),
    scratch_types=[
        pltpu.VMEM((NCH, IPC), jnp.int32),
        pltpu.VMEM((IPC, D), jnp.float32),
        pltpu.VMEM((BPW, D), jnp.float32),
        pltpu.SemaphoreType.DMA,
    ],
)
def _mean_embed(table_hbm, xr_hbm, out_hbm, idx_v, rows_v, out_v, sem):
    wid = lax.axis_index("s") * 2 + lax.axis_index("c")
    pltpu.sync_copy(xr_hbm.at[wid], idx_v)

    def chunk_body(c, carry):
        pltpu.async_copy(table_hbm.at[idx_v.at[c]], rows_v, sem).wait()
        for r in range(RPC):
            def red(l, accs):
                return tuple(accs[j] + rows_v[r * L + l, pl.ds(j * 16, 16)]
                             for j in range(NJ))
            accs = lax.fori_loop(
                0, L, red,
                tuple(jnp.zeros((16,), jnp.float32) for _ in range(NJ)))
            for j in range(NJ):
                out_v[c * RPC + r, pl.ds(j * 16, 16)] = accs[j] * (1.0 / L)
        return carry

    lax.fori_loop(0, NCH, chunk_body, 0)
    pltpu.sync_copy(out_v, out_hbm.at[pl.ds(wid * BPW, BPW)])


def kernel(vectors, x):
    xr = x.astype(jnp.int32).reshape(NW, NCH, IPC)
    return _mean_embed(vectors, xr)


# speedup vs baseline: 8.4789x; 1.2849x over previous
import functools

import jax
import jax.numpy as jnp
from jax import lax
from jax.experimental import pallas as pl
from jax.experimental.pallas import tpu as pltpu
from jax.experimental.pallas import tpu_sc as plsc

B = 4096
L = 50
D = 64
NW = 32
BPW = B // NW
RPC = 2
NCH = BPW // RPC
IPC = RPC * L
NJ = D // 16
NBUF = 2

_mesh = plsc.VectorSubcoreMesh(core_axis_name="c", subcore_axis_name="s")


@functools.partial(
    pl.kernel,
    mesh=_mesh,
    compiler_params=pltpu.CompilerParams(use_tc_tiling_on_sc=False),
    out_type=jax.ShapeDtypeStruct((B, D), jnp.float32),
    scratch_types=[
        pltpu.VMEM((NCH, IPC), jnp.int32),
        pltpu.VMEM((NBUF, IPC, D), jnp.float32),
        pltpu.VMEM((BPW, D), jnp.float32),
        pltpu.SemaphoreType.DMA,
        pltpu.SemaphoreType.DMA,
    ],
)
def _mean_embed(table_hbm, xr_hbm, out_hbm, idx_v, rows_v, out_v, sem0, sem1):
    wid = lax.axis_index("s") * 2 + lax.axis_index("c")
    pltpu.sync_copy(xr_hbm.at[wid], idx_v)
    sems = (sem0, sem1)

    def gather(c, b):
        return pltpu.make_async_copy(
            table_hbm.at[idx_v.at[c]], rows_v.at[b], sems[b])

    gather(0, 0).start()
    gather(1, 1).start()

    def step(i, carry):
        for b in range(NBUF):
            c = NBUF * i + b
            gather(c, b).wait()
            for r in range(RPC):
                base = r * L
                acc = [None] * (2 * NJ)
                for l in range(L):
                    for j in range(NJ):
                        v = rows_v[b, base + l, pl.ds(j * 16, 16)]
                        k = (l % 2) * NJ + j
                        acc[k] = v if acc[k] is None else acc[k] + v
                for j in range(NJ):
                    out_v[c * RPC + r, pl.ds(j * 16, 16)] = (
                        (acc[j] + acc[NJ + j]) * (1.0 / L))

            @pl.when(c + NBUF < NCH)
            def _():
                gather(c + NBUF, b).start()
        return carry

    lax.fori_loop(0, NCH // NBUF, step, 0)
    pltpu.sync_copy(out_v, out_hbm.at[pl.ds(wid * BPW, BPW)])


def kernel(vectors, x):
    xr = x.astype(jnp.int32).reshape(NW, NCH, IPC)
    return _mean_embed(vectors, xr)
